# baseline (speedup 1.0000x reference)
import jax
import jax.numpy as jnp
from jax.experimental import pallas as pl
from jax.experimental.pallas import tpu as pltpu
from jax.experimental.pallas import tpu_sc as plsc

_W = 128
_TR = 5200
_S = 3


def _sc_gather(idx, pe, rows, D):
    mesh = plsc.VectorSubcoreMesh(core_axis_name="c", subcore_axis_name="s")

    @pl.kernel(out_type=jax.ShapeDtypeStruct((rows, D), jnp.float32), mesh=mesh)
    def gather_k(i_hbm, pe_hbm, g_hbm):
        def body(i_vmem, g_vmem):
            pltpu.sync_copy(pe_hbm.at[i_vmem.at[0]], g_vmem)

        pltpu.emit_pipeline(
            body,
            grid=(rows // _W,),
            in_specs=[pl.BlockSpec((1, _W), lambda i: (0, i))],
            out_specs=[pl.BlockSpec((_W, D), lambda i: (i, 0))],
            core_axis_name=("c", "s"),
            dimension_semantics=(pltpu.PARALLEL,),
        )(i_hbm, g_hbm)

    return gather_k(idx, pe)


def _tc_add_chunk(x, g, prev, chunk, R, D):
    rows = R // _S
    blocks = rows // _TR
    base = chunk * blocks

    in_specs = [
        pl.BlockSpec((_TR, D), lambda i: (base + i, 0)),
        pl.BlockSpec((_TR, D), lambda i: (i, 0)),
    ]
    operands = [x, g]
    aliases = {}
    if prev is not None:
        in_specs.append(pl.BlockSpec(memory_space=pltpu.MemorySpace.HBM))
        operands.append(prev)
        aliases = {2: 0}

    def add_k(x_ref, g_ref, *rest):
        o_ref = rest[-1]
        o_ref[...] = x_ref[...] + g_ref[...]

    return pl.pallas_call(
        add_k,
        grid=(blocks,),
        in_specs=in_specs,
        out_specs=pl.BlockSpec((_TR, D), lambda i: (base + i, 0)),
        out_shape=jax.ShapeDtypeStruct((R, D), jnp.float32),
        input_output_aliases=aliases,
    )(*operands)


def kernel(input_emb, position, pe):
    B, N, L, D = input_emb.shape
    R = B * N * L
    rows = R // _S

    @jax.jit
    def run(input_emb, position, pe):
        x = input_emb.transpose(1, 2, 0, 3).reshape(R, D)
        idx = position.transpose(1, 2, 0).reshape(1, R).astype(jnp.int32)
        gs = []
        prev_g = None
        for c in range(_S):
            idx_c = jax.lax.slice(idx, (0, c * rows), (1, (c + 1) * rows))
            if prev_g is not None:
                idx_c, _ = jax.lax.optimization_barrier((idx_c, prev_g))
            prev_g = _sc_gather(idx_c, pe, rows, D)
            gs.append(prev_g)
        out = None
        for c in range(_S):
            out = _tc_add_chunk(x, gs[c], out, c, R, D)
        return out.reshape(N, L, B, D).transpose(2, 0, 1, 3)

    return run(input_emb, position, pe)


# speedup vs baseline: 1.0434x; 1.0434x over previous
import jax
import jax.numpy as jnp
from jax.experimental import pallas as pl
from jax.experimental.pallas import tpu as pltpu
from jax.experimental.pallas import tpu_sc as plsc

_W = 128
_TR = 7800


def _sc_gather(idx, pe, R, D):
    mesh = plsc.VectorSubcoreMesh(core_axis_name="c", subcore_axis_name="s")

    @pl.kernel(out_type=jax.ShapeDtypeStruct((R, D), jnp.float32), mesh=mesh)
    def gather_k(i_hbm, pe_hbm, g_hbm):
        def body(i_vmem, g_vmem):
            pltpu.sync_copy(pe_hbm.at[i_vmem.at[0]], g_vmem)

        pltpu.emit_pipeline(
            body,
            grid=(R // _W,),
            in_specs=[pl.BlockSpec((1, _W), lambda i: (0, i))],
            out_specs=[pl.BlockSpec((_W, D), lambda i: (i, 0))],
            core_axis_name=("c", "s"),
            dimension_semantics=(pltpu.PARALLEL,),
        )(i_hbm, g_hbm)

    return gather_k(idx, pe)


def _tc_add(x, g, R, D):
    def add_k(x_ref, g_ref, o_ref):
        o_ref[...] = x_ref[...] + g_ref[...]

    return pl.pallas_call(
        add_k,
        grid=(R // _TR,),
        in_specs=[
            pl.BlockSpec((_TR, D), lambda i: (i, 0)),
            pl.BlockSpec((_TR, D), lambda i: (i, 0)),
        ],
        out_specs=pl.BlockSpec((_TR, D), lambda i: (i, 0)),
        out_shape=jax.ShapeDtypeStruct((R, D), jnp.float32),
    )(x, g)


def kernel(input_emb, position, pe):
    B, N, L, D = input_emb.shape
    R = B * N * L

    @jax.jit
    def run(input_emb, position, pe):
        x = input_emb.transpose(1, 2, 0, 3).reshape(R, D)
        idx = position.transpose(1, 2, 0).reshape(1, R).astype(jnp.int32)
        g = _sc_gather(idx, pe, R, D)
        out = _tc_add(x, g, R, D)
        return out.reshape(N, L, B, D).transpose(2, 0, 1, 3)

    return run(input_emb, position, pe)
